# initial kernel scaffold (unmeasured)
import jax
import jax.numpy as jnp
from jax import lax
from jax.experimental import pallas as pl
from jax.experimental.pallas import tpu as pltpu

N_DEV = 32
B = 2
SQ = 256
SKV = 256
HQ_PER = 4
DH = 64
D_MODEL = 512
ROWS = B * SQ
BLK = ROWS // N_DEV

RS_MASKS = (16, 8, 4, 2, 1)
AG_MASKS = (1, 2, 4, 8, 16)
RS_SIZES = (256, 128, 64, 32, 16)
RS_OFFS = (0, 256, 384, 448, 480)


def kernel(x, Wq, K_ext, V_ext, Wo):
    def body(x_ref, wq_ref, k_hbm, v_hbm, wo_ref, out_ref,
             acc, recvbuf, k_vmem, v_vmem,
             kv_sems, rs_send, rs_recv, ag_send, ag_recv):
        my = lax.axis_index("i")

        barrier = pltpu.get_barrier_semaphore()
        for m in RS_MASKS:
            pl.semaphore_signal(
                barrier, inc=1,
                device_id=(my ^ m,), device_id_type=pl.DeviceIdType.MESH)
        pl.semaphore_wait(barrier, len(RS_MASKS))

        h0 = my * HQ_PER
        k_copy = pltpu.make_async_copy(
            k_hbm.at[:, :, pl.ds(h0, HQ_PER), :], k_vmem, kv_sems.at[0])
        v_copy = pltpu.make_async_copy(
            v_hbm.at[:, :, pl.ds(h0, HQ_PER), :], v_vmem, kv_sems.at[1])
        k_copy.start()
        v_copy.start()

        xf = x_ref[...].reshape(ROWS, D_MODEL)
        qf = jnp.dot(xf, wq_ref[...], preferred_element_type=jnp.float32)

        qi = lax.broadcasted_iota(jnp.int32, (SQ, SKV), 0)
        ki = lax.broadcasted_iota(jnp.int32, (SQ, SKV), 1)
        mask = (jnp.abs(qi - ki) <= 128) | (ki < 32) | (qi < 32)

        k_copy.wait()
        v_copy.wait()

        ctx_rows = []
        for b in range(B):
            heads = []
            for h in range(HQ_PER):
                q_bh = qf[b * SQ:(b + 1) * SQ, h * DH:(h + 1) * DH]
                k_bh = k_vmem[b, :, h, :]
                v_bh = v_vmem[b, :, h, :]
                s = jnp.dot(q_bh, k_bh.T, preferred_element_type=jnp.float32)
                s = jnp.where(mask, s * 0.125, -1e9)
                s = s - jnp.max(s, axis=-1, keepdims=True)
                w = jnp.exp(s)
                w = w / jnp.sum(w, axis=-1, keepdims=True)
                heads.append(
                    jnp.dot(w, v_bh, preferred_element_type=jnp.float32))
            ctx_rows.append(jnp.concatenate(heads, axis=-1))
        ctxf = jnp.concatenate(ctx_rows, axis=0)

        acc[...] = jnp.dot(ctxf, wo_ref[...],
                           preferred_element_type=jnp.float32)

        lo = jnp.int32(0)
        for k, m in enumerate(RS_MASKS):
            half = RS_SIZES[k]
            bit = (my & m) != 0
            send_start = jnp.where(bit, lo, lo + half)
            new_lo = jnp.where(bit, lo + half, lo)
            rdma = pltpu.make_async_remote_copy(
                src_ref=acc.at[pl.ds(send_start, half)],
                dst_ref=recvbuf.at[pl.ds(RS_OFFS[k], half)],
                send_sem=rs_send.at[k], recv_sem=rs_recv.at[k],
                device_id=(my ^ m,), device_id_type=pl.DeviceIdType.MESH)
            rdma.start()
            rdma.wait_recv()
            acc[pl.ds(new_lo, half), :] = (
                acc[pl.ds(new_lo, half), :]
                + recvbuf[RS_OFFS[k]:RS_OFFS[k] + half, :])
            rdma.wait_send()
            lo = new_lo

        for j, m in enumerate(AG_MASKS):
            size = BLK * m
            glo = (my & ~(m - 1)) * BLK
            rdma = pltpu.make_async_remote_copy(
                src_ref=acc.at[pl.ds(glo, size)],
                dst_ref=acc.at[pl.ds(glo, size)],
                send_sem=ag_send.at[j], recv_sem=ag_recv.at[j],
                device_id=(my ^ m,), device_id_type=pl.DeviceIdType.MESH)
            rdma.start()
            rdma.wait_recv()
            rdma.wait_send()

        for b in range(B):
            out_ref[b] = acc[b * SQ:(b + 1) * SQ, :]

    return pl.pallas_call(
        body,
        out_shape=jax.ShapeDtypeStruct((B, SQ, D_MODEL), jnp.float32),
        in_specs=[
            pl.BlockSpec(memory_space=pltpu.VMEM),
            pl.BlockSpec(memory_space=pltpu.VMEM),
            pl.BlockSpec(memory_space=pltpu.ANY),
            pl.BlockSpec(memory_space=pltpu.ANY),
            pl.BlockSpec(memory_space=pltpu.VMEM),
        ],
        out_specs=pl.BlockSpec(memory_space=pltpu.VMEM),
        scratch_shapes=[
            pltpu.VMEM((ROWS, D_MODEL), jnp.float32),
            pltpu.VMEM((ROWS, D_MODEL), jnp.float32),
            pltpu.VMEM((B, SKV, HQ_PER, DH), jnp.float32),
            pltpu.VMEM((B, SKV, HQ_PER, DH), jnp.float32),
            pltpu.SemaphoreType.DMA((2,)),
            pltpu.SemaphoreType.DMA((len(RS_MASKS),)),
            pltpu.SemaphoreType.DMA((len(RS_MASKS),)),
            pltpu.SemaphoreType.DMA((len(AG_MASKS),)),
            pltpu.SemaphoreType.DMA((len(AG_MASKS),)),
        ],
        compiler_params=pltpu.CompilerParams(collective_id=0),
    )(x, Wq, K_ext, V_ext, Wo)


# baseline (device time: 107377 ns/iter reference)
import jax
import jax.numpy as jnp
from jax import lax
from jax.experimental import pallas as pl
from jax.experimental.pallas import tpu as pltpu

N_DEV = 32
B = 2
SQ = 256
SKV = 256
HQ_PER = 4
DH = 64
D_MODEL = 512
ROWS = B * SQ
BLK = ROWS // N_DEV

RS_MASKS = (16, 8, 4, 2, 1)
AG_MASKS = (1, 2, 4, 8, 16)
RS_SIZES = (256, 128, 64, 32, 16)
RS_OFFS = (0, 256, 384, 448, 480)


def kernel(x, Wq, K_ext, V_ext, Wo):
    def body(x_ref, wq_ref, k_hbm, v_hbm, wo_ref, out_ref,
             acc, recvbuf, k_vmem, v_vmem,
             kv_sems, rs_send, rs_recv, ag_send, ag_recv):
        my = lax.axis_index("i")

        barrier = pltpu.get_barrier_semaphore()
        for m in RS_MASKS:
            pl.semaphore_signal(
                barrier, inc=1,
                device_id=(my ^ m,), device_id_type=pl.DeviceIdType.MESH)
        pl.semaphore_wait(barrier, len(RS_MASKS))

        h0 = my * HQ_PER
        k_copy = pltpu.make_async_copy(
            k_hbm.at[:, :, pl.ds(h0, HQ_PER), :], k_vmem, kv_sems.at[0])
        v_copy = pltpu.make_async_copy(
            v_hbm.at[:, :, pl.ds(h0, HQ_PER), :], v_vmem, kv_sems.at[1])
        k_copy.start()
        v_copy.start()

        xf = x_ref[...].reshape(ROWS, D_MODEL)
        qf = jnp.dot(xf, wq_ref[...], preferred_element_type=jnp.float32)

        qi = lax.broadcasted_iota(jnp.int32, (SQ, SKV), 0)
        ki = lax.broadcasted_iota(jnp.int32, (SQ, SKV), 1)
        mask = (jnp.abs(qi - ki) <= 128) | (ki < 32) | (qi < 32)

        k_copy.wait()
        v_copy.wait()

        ctx_rows = []
        for b in range(B):
            heads = []
            for h in range(HQ_PER):
                q_bh = qf[b * SQ:(b + 1) * SQ, h * DH:(h + 1) * DH]
                k_bh = k_vmem[b, :, h, :]
                v_bh = v_vmem[b, :, h, :]
                s = jnp.dot(q_bh, k_bh.T, preferred_element_type=jnp.float32)
                s = jnp.where(mask, s * 0.125, -1e9)
                s = s - jnp.max(s, axis=-1, keepdims=True)
                w = jnp.exp(s)
                w = w / jnp.sum(w, axis=-1, keepdims=True)
                heads.append(
                    jnp.dot(w, v_bh, preferred_element_type=jnp.float32))
            ctx_rows.append(jnp.concatenate(heads, axis=-1))
        ctxf = jnp.concatenate(ctx_rows, axis=0)

        acc[...] = jnp.dot(ctxf, wo_ref[...],
                           preferred_element_type=jnp.float32)

        lo = jnp.int32(0)
        for k, m in enumerate(RS_MASKS):
            half = RS_SIZES[k]
            bit = (my & m) != 0
            send_start = pl.multiple_of(jnp.where(bit, lo, lo + half), BLK)
            new_lo = pl.multiple_of(jnp.where(bit, lo + half, lo), BLK)
            rdma = pltpu.make_async_remote_copy(
                src_ref=acc.at[pl.ds(send_start, half)],
                dst_ref=recvbuf.at[pl.ds(RS_OFFS[k], half)],
                send_sem=rs_send.at[k], recv_sem=rs_recv.at[k],
                device_id=(my ^ m,), device_id_type=pl.DeviceIdType.MESH)
            rdma.start()
            rdma.wait_recv()
            acc[pl.ds(new_lo, half), :] = (
                acc[pl.ds(new_lo, half), :]
                + recvbuf[RS_OFFS[k]:RS_OFFS[k] + half, :])
            rdma.wait_send()
            lo = new_lo

        for j, m in enumerate(AG_MASKS):
            size = BLK * m
            glo = pl.multiple_of((my & ~(m - 1)) * BLK, BLK)
            rdma = pltpu.make_async_remote_copy(
                src_ref=acc.at[pl.ds(glo, size)],
                dst_ref=acc.at[pl.ds(glo, size)],
                send_sem=ag_send.at[j], recv_sem=ag_recv.at[j],
                device_id=(my ^ m,), device_id_type=pl.DeviceIdType.MESH)
            rdma.start()
            rdma.wait_recv()
            rdma.wait_send()

        for b in range(B):
            out_ref[b] = acc[b * SQ:(b + 1) * SQ, :]

    return pl.pallas_call(
        body,
        out_shape=jax.ShapeDtypeStruct((B, SQ, D_MODEL), jnp.float32),
        in_specs=[
            pl.BlockSpec(memory_space=pltpu.VMEM),
            pl.BlockSpec(memory_space=pltpu.VMEM),
            pl.BlockSpec(memory_space=pltpu.MemorySpace.HBM),
            pl.BlockSpec(memory_space=pltpu.MemorySpace.HBM),
            pl.BlockSpec(memory_space=pltpu.VMEM),
        ],
        out_specs=pl.BlockSpec(memory_space=pltpu.VMEM),
        scratch_shapes=[
            pltpu.VMEM((ROWS, D_MODEL), jnp.float32),
            pltpu.VMEM((ROWS, D_MODEL), jnp.float32),
            pltpu.VMEM((B, SKV, HQ_PER, DH), jnp.float32),
            pltpu.VMEM((B, SKV, HQ_PER, DH), jnp.float32),
            pltpu.SemaphoreType.DMA((2,)),
            pltpu.SemaphoreType.DMA((len(RS_MASKS),)),
            pltpu.SemaphoreType.DMA((len(RS_MASKS),)),
            pltpu.SemaphoreType.DMA((len(AG_MASKS),)),
            pltpu.SemaphoreType.DMA((len(AG_MASKS),)),
        ],
        compiler_params=pltpu.CompilerParams(collective_id=0),
    )(x, Wq, K_ext, V_ext, Wo)


# device time: 85208 ns/iter; 1.2602x vs baseline; 1.2602x over previous
import jax
import jax.numpy as jnp
from jax import lax
from jax.experimental import pallas as pl
from jax.experimental.pallas import tpu as pltpu

N_DEV = 32
B = 2
SQ = 256
SKV = 256
HQ_PER = 4
DH = 64
D_MODEL = 512
ROWS = B * SQ
BLK = ROWS // N_DEV


def kernel(x, Wq, K_ext, V_ext, Wo):
    def body(x_ref, wq_ref, k_hbm, v_hbm, wo_ref, out_ref,
             acc, rs_slots, k_vmem, v_vmem,
             kv_sems, rs_send, rs_recv, ag_send, ag_recv):
        my = lax.axis_index("i")

        barrier = pltpu.get_barrier_semaphore()
        for d in range(1, N_DEV):
            pl.semaphore_signal(
                barrier, inc=1,
                device_id=((my + d) % N_DEV,),
                device_id_type=pl.DeviceIdType.MESH)
        pl.semaphore_wait(barrier, N_DEV - 1)

        h0 = my * HQ_PER
        k_copy = pltpu.make_async_copy(
            k_hbm.at[:, :, pl.ds(h0, HQ_PER), :], k_vmem, kv_sems.at[0])
        v_copy = pltpu.make_async_copy(
            v_hbm.at[:, :, pl.ds(h0, HQ_PER), :], v_vmem, kv_sems.at[1])
        k_copy.start()
        v_copy.start()

        xf = x_ref[...].reshape(ROWS, D_MODEL)
        qf = jnp.dot(xf, wq_ref[...], preferred_element_type=jnp.float32)

        qi = lax.broadcasted_iota(jnp.int32, (SQ, SKV), 0)
        ki = lax.broadcasted_iota(jnp.int32, (SQ, SKV), 1)
        mask = (jnp.abs(qi - ki) <= 128) | (ki < 32) | (qi < 32)

        k_copy.wait()
        v_copy.wait()

        ctx_rows = []
        for b in range(B):
            heads = []
            for h in range(HQ_PER):
                q_bh = qf[b * SQ:(b + 1) * SQ, h * DH:(h + 1) * DH]
                k_bh = k_vmem[b, :, h, :]
                v_bh = v_vmem[b, :, h, :]
                s = jnp.dot(q_bh, k_bh.T, preferred_element_type=jnp.float32)
                s = jnp.where(mask, s * 0.125, -1e9)
                s = s - jnp.max(s, axis=-1, keepdims=True)
                w = jnp.exp(s)
                w = w / jnp.sum(w, axis=-1, keepdims=True)
                heads.append(
                    jnp.dot(w, v_bh, preferred_element_type=jnp.float32))
            ctx_rows.append(jnp.concatenate(heads, axis=-1))
        ctxf = jnp.concatenate(ctx_rows, axis=0)

        acc[...] = jnp.dot(ctxf, wo_ref[...],
                           preferred_element_type=jnp.float32)

        myblk = pl.multiple_of(my * BLK, BLK)

        for d in range(1, N_DEV):
            peer = (my + d) % N_DEV
            pblk = pl.multiple_of(peer * BLK, BLK)
            rdma = pltpu.make_async_remote_copy(
                src_ref=acc.at[pl.ds(pblk, BLK)],
                dst_ref=rs_slots.at[pl.ds(myblk, BLK)],
                send_sem=rs_send.at[d],
                recv_sem=rs_recv.at[my],
                device_id=(peer,),
                device_id_type=pl.DeviceIdType.MESH)
            rdma.start()

        red = acc[pl.ds(myblk, BLK), :]
        for d in range(1, N_DEV):
            s = (my + d) % N_DEV
            sblk = pl.multiple_of(s * BLK, BLK)
            recv = pltpu.make_async_remote_copy(
                src_ref=rs_slots.at[pl.ds(sblk, BLK)],
                dst_ref=rs_slots.at[pl.ds(sblk, BLK)],
                send_sem=rs_send.at[d],
                recv_sem=rs_recv.at[s],
                device_id=(s,),
                device_id_type=pl.DeviceIdType.MESH)
            recv.wait_recv()
            red = red + rs_slots[pl.ds(sblk, BLK), :]
        acc[pl.ds(myblk, BLK), :] = red

        for d in range(1, N_DEV):
            peer = (my + d) % N_DEV
            rdma = pltpu.make_async_remote_copy(
                src_ref=acc.at[pl.ds(myblk, BLK)],
                dst_ref=acc.at[pl.ds(myblk, BLK)],
                send_sem=ag_send.at[d],
                recv_sem=ag_recv.at[my],
                device_id=(peer,),
                device_id_type=pl.DeviceIdType.MESH)
            rdma.start()

        for d in range(1, N_DEV):
            s = (my + d) % N_DEV
            sblk = pl.multiple_of(s * BLK, BLK)
            recv = pltpu.make_async_remote_copy(
                src_ref=acc.at[pl.ds(sblk, BLK)],
                dst_ref=acc.at[pl.ds(sblk, BLK)],
                send_sem=ag_send.at[d],
                recv_sem=ag_recv.at[s],
                device_id=(s,),
                device_id_type=pl.DeviceIdType.MESH)
            recv.wait_recv()

        for b in range(B):
            out_ref[b] = acc[b * SQ:(b + 1) * SQ, :]

        for d in range(1, N_DEV):
            peer = (my + d) % N_DEV
            pblk = pl.multiple_of(peer * BLK, BLK)
            pltpu.make_async_remote_copy(
                src_ref=acc.at[pl.ds(pblk, BLK)],
                dst_ref=rs_slots.at[pl.ds(myblk, BLK)],
                send_sem=rs_send.at[d],
                recv_sem=rs_recv.at[my],
                device_id=(peer,),
                device_id_type=pl.DeviceIdType.MESH).wait_send()
            pltpu.make_async_remote_copy(
                src_ref=acc.at[pl.ds(myblk, BLK)],
                dst_ref=acc.at[pl.ds(myblk, BLK)],
                send_sem=ag_send.at[d],
                recv_sem=ag_recv.at[my],
                device_id=(peer,),
                device_id_type=pl.DeviceIdType.MESH).wait_send()

    return pl.pallas_call(
        body,
        out_shape=jax.ShapeDtypeStruct((B, SQ, D_MODEL), jnp.float32),
        in_specs=[
            pl.BlockSpec(memory_space=pltpu.MemorySpace.VMEM),
            pl.BlockSpec(memory_space=pltpu.MemorySpace.VMEM),
            pl.BlockSpec(memory_space=pltpu.MemorySpace.HBM),
            pl.BlockSpec(memory_space=pltpu.MemorySpace.HBM),
            pl.BlockSpec(memory_space=pltpu.MemorySpace.VMEM),
        ],
        out_specs=pl.BlockSpec(memory_space=pltpu.MemorySpace.VMEM),
        scratch_shapes=[
            pltpu.VMEM((ROWS, D_MODEL), jnp.float32),
            pltpu.VMEM((ROWS, D_MODEL), jnp.float32),
            pltpu.VMEM((B, SKV, HQ_PER, DH), jnp.float32),
            pltpu.VMEM((B, SKV, HQ_PER, DH), jnp.float32),
            pltpu.SemaphoreType.DMA((2,)),
            pltpu.SemaphoreType.DMA((N_DEV,)),
            pltpu.SemaphoreType.DMA((N_DEV,)),
            pltpu.SemaphoreType.DMA((N_DEV,)),
            pltpu.SemaphoreType.DMA((N_DEV,)),
        ],
        compiler_params=pltpu.CompilerParams(collective_id=0),
    )(x, Wq, K_ext, V_ext, Wo)
